# baseline (speedup 1.0000x reference)
import functools

import jax
import jax.numpy as jnp
from jax import lax
from jax.experimental import pallas as pl
from jax.experimental.pallas import tpu as pltpu
from jax.experimental.pallas import tpu_sc as plsc

NUM_CORES = 2
NUM_SUBCORES = 16
NUM_WORKERS = NUM_CORES * NUM_SUBCORES
LANE = 128
NBUF = 16

_DNUMS = lax.GatherDimensionNumbers(
    offset_dims=(), collapsed_slice_dims=(0,), start_index_map=(0,)
)


def kernel(inputs, table):
    idx = inputs.astype(jnp.int32)
    (batch,) = idx.shape
    vocab, dim = table.shape
    assert batch % NUM_WORKERS == 0
    b_per_w = batch // NUM_WORKERS
    assert b_per_w % NBUF == 0
    idx2 = idx.reshape(NUM_WORKERS, b_per_w)
    table_t = table.T

    mesh = plsc.VectorSubcoreMesh(
        core_axis_name="c",
        subcore_axis_name="s",
        num_cores=NUM_CORES,
        num_subcores=NUM_SUBCORES,
    )

    @functools.partial(
        pl.kernel,
        mesh=mesh,
        out_type=jax.ShapeDtypeStruct((batch * dim,), jnp.float32),
        scratch_types=[
            pltpu.VMEM((b_per_w + 16,), jnp.int32),
            pltpu.VMEM((NBUF, dim, LANE), jnp.float32),
            pltpu.VMEM((b_per_w * dim,), jnp.float32),
            [pltpu.SemaphoreType.DMA] * NBUF,
        ],
    )
    def gather_kernel(table_hbm, idx_hbm, out_hbm, idx_v, slab_v, rows_v, sems):
        wid = lax.axis_index("s") * NUM_CORES + lax.axis_index("c")
        pltpu.sync_copy(idx_hbm.at[wid], idx_v.at[pl.ds(0, b_per_w)])
        row_iota = lax.iota(jnp.int32, 16)

        def slab_copy(k, buf):
            j = idx_v[pl.ds(k, 16)][0] // LANE
            pltpu.async_copy(
                table_hbm.at[:, pl.ds(pl.multiple_of(j * LANE, LANE), LANE)],
                slab_v.at[buf],
                sems[buf],
            )

        for b in range(NBUF):
            slab_copy(b, b)

        def body(g, carry):
            k0 = g * NBUF
            for buf in range(NBUF):
                k = k0 + buf
                pltpu.make_async_copy(
                    table_hbm.at[:, pl.ds(0, LANE)], slab_v.at[buf], sems[buf]
                ).wait()
                m = lax.rem(idx_v[pl.ds(k, 16)][0], LANE)
                m_f = pl.multiple_of((m // 16) * 16, 16)
                d16 = jnp.full((16, 1), m - m_f, jnp.int32)
                acc_lo = jnp.zeros((16,), jnp.float32)
                acc_hi = jnp.zeros((16,), jnp.float32)
                for c in range(dim):
                    v_c = slab_v[buf, c, pl.ds(m_f, 16)]
                    splat = lax.gather(
                        v_c, d16, _DNUMS, (1,),
                        mode=lax.GatherScatterMode.PROMISE_IN_BOUNDS,
                    )
                    if c < 16:
                        acc_lo = jnp.where(row_iota == c, splat, acc_lo)
                    else:
                        acc_hi = jnp.where(row_iota == c - 16, splat, acc_hi)
                rows_v[pl.ds(k * dim, 16)] = acc_lo
                rows_v[pl.ds(k * dim + 16, 16)] = acc_hi

                @pl.when(k + NBUF < b_per_w)
                def _():
                    slab_copy(k + NBUF, buf)

            return carry

        lax.fori_loop(0, b_per_w // NBUF, body, 0)
        pltpu.sync_copy(
            rows_v, out_hbm.at[pl.ds(wid * b_per_w * dim, b_per_w * dim)]
        )

    return gather_kernel(table_t, idx2).reshape(batch, dim)


# speedup vs baseline: 1.0217x; 1.0217x over previous
import functools

import jax
import jax.numpy as jnp
from jax import lax
from jax.experimental import pallas as pl
from jax.experimental.pallas import tpu as pltpu
from jax.experimental.pallas import tpu_sc as plsc

NUM_CORES = 2
NUM_SUBCORES = 16
NUM_WORKERS = NUM_CORES * NUM_SUBCORES
LANE = 128
NBUF = 8

_DNUMS = lax.GatherDimensionNumbers(
    offset_dims=(), collapsed_slice_dims=(0,), start_index_map=(0,)
)


def kernel(inputs, table):
    idx = inputs.astype(jnp.int32)
    (batch,) = idx.shape
    vocab, dim = table.shape
    assert batch % NUM_WORKERS == 0
    b_per_w = batch // NUM_WORKERS
    assert b_per_w % NBUF == 0
    idx2 = idx.reshape(NUM_WORKERS, b_per_w)
    table_t = table.T

    mesh = plsc.VectorSubcoreMesh(
        core_axis_name="c",
        subcore_axis_name="s",
        num_cores=NUM_CORES,
        num_subcores=NUM_SUBCORES,
    )

    @functools.partial(
        pl.kernel,
        mesh=mesh,
        out_type=jax.ShapeDtypeStruct((batch * dim,), jnp.float32),
        scratch_types=[
            pltpu.VMEM((b_per_w + 16,), jnp.int32),
            pltpu.VMEM((NBUF, dim, LANE), jnp.float32),
            pltpu.VMEM((b_per_w * dim,), jnp.float32),
            [pltpu.SemaphoreType.DMA] * NBUF,
        ],
    )
    def gather_kernel(table_hbm, idx_hbm, out_hbm, idx_v, slab_v, rows_v, sems):
        wid = lax.axis_index("s") * NUM_CORES + lax.axis_index("c")
        pltpu.sync_copy(idx_hbm.at[wid], idx_v.at[pl.ds(0, b_per_w)])
        row_iota = lax.iota(jnp.int32, 16)

        def slab_copy(k, buf):
            j = idx_v[pl.ds(k, 16)][0] // LANE
            pltpu.async_copy(
                table_hbm.at[:, pl.ds(pl.multiple_of(j * LANE, LANE), LANE)],
                slab_v.at[buf],
                sems[buf],
            )

        for b in range(NBUF):
            slab_copy(b, b)

        def body(g, carry):
            k0 = g * NBUF
            for buf in range(NBUF):
                k = k0 + buf
                pltpu.make_async_copy(
                    table_hbm.at[:, pl.ds(0, LANE)], slab_v.at[buf], sems[buf]
                ).wait()
                m = lax.rem(idx_v[pl.ds(k, 16)][0], LANE)
                m_f = pl.multiple_of((m // 16) * 16, 16)
                d16 = jnp.full((16, 1), m - m_f, jnp.int32)
                acc_lo = jnp.zeros((16,), jnp.float32)
                acc_hi = jnp.zeros((16,), jnp.float32)
                for c in range(dim):
                    v_c = slab_v[buf, c, pl.ds(m_f, 16)]
                    splat = lax.gather(
                        v_c, d16, _DNUMS, (1,),
                        mode=lax.GatherScatterMode.PROMISE_IN_BOUNDS,
                    )
                    if c < 16:
                        acc_lo = jnp.where(row_iota == c, splat, acc_lo)
                    else:
                        acc_hi = jnp.where(row_iota == c - 16, splat, acc_hi)
                rows_v[pl.ds(k * dim, 16)] = acc_lo
                rows_v[pl.ds(k * dim + 16, 16)] = acc_hi

                @pl.when(k + NBUF < b_per_w)
                def _():
                    slab_copy(k + NBUF, buf)

            return carry

        lax.fori_loop(0, b_per_w // NBUF, body, 0)
        pltpu.sync_copy(
            rows_v, out_hbm.at[pl.ds(wid * b_per_w * dim, b_per_w * dim)]
        )

    return gather_kernel(table_t, idx2).reshape(batch, dim)
